# baseline (speedup 1.0000x reference)
import functools

import jax
import jax.numpy as jnp
from jax import lax
from jax.experimental import pallas as pl
from jax.experimental.pallas import tpu as pltpu
from jax.experimental.pallas import tpu_sc as plsc

NU = 50000
NG = 5000
NI = 20000
EMB = 64
NHG = NU + NG
NGI = NG + NI
EHG = 880000
EGI = 400000
BATCH = 4096

NC = 2
NS = 16

NHG_PAD = 55296
ACC_CS = 55312
HG_LOOP = 432
HG_TCH = 434

GI_ACC = 5008
GI_LOOP = 196
GI_TCH = 198

_mesh = plsc.VectorSubcoreMesh(core_axis_name="c", subcore_axis_name="s")


def _zero_fill(zbuf, nq=4):
    zv = jnp.zeros((16,), jnp.float32)
    for i in range(128):
        for q in range(nq):
            zbuf[i, pl.ds(q * 16, 16)] = zv


def _scale_rows(rows_v, vb, nq=4):
    def scale_body(g, carry2):
        e0 = g * 16
        vv = vb[0, pl.ds(g * 16, 16)]
        for t in range(16):
            e = e0 + t
            vs = vv.at[jnp.full((16,), t, jnp.int32)].get(
                mode="promise_in_bounds")
            for q in range(nq):
                sl = rows_v[e, pl.ds(q * 16, 16)]
                rows_v[e, pl.ds(q * 16, 16)] = sl * vs
        return carry2

    lax.fori_loop(0, 8, scale_body, 0)


def _spmm_colsplit(x, meta2d, val2d, out_half, acc_rows, loop, tch):
    zrows = acc_rows // NS
    z_full, z_rem = divmod(zrows, 128)
    orows = out_half // NS
    o_full, o_rem = divmod(orows, 128)

    @functools.partial(
        pl.kernel,
        mesh=_mesh,
        compiler_params=pltpu.CompilerParams(use_tc_tiling_on_sc=False),
        out_type=jax.ShapeDtypeStruct((2 * out_half, 32), jnp.float32),
        scratch_types=[
            pltpu.VMEM_SHARED((acc_rows, 32), jnp.float32),
            pltpu.VMEM((3, 128), jnp.int32),
            pltpu.VMEM((3, 128), jnp.int32),
            pltpu.VMEM((1, 128), jnp.float32),
            pltpu.VMEM((1, 128), jnp.float32),
            pltpu.VMEM((128, 32), jnp.float32),
            pltpu.VMEM((128, 32), jnp.float32),
            pltpu.VMEM((1, 128), jnp.int32),
            pltpu.VMEM((1, 128), jnp.int32),
            pltpu.SemaphoreType.DMA,
            pltpu.SemaphoreType.DMA,
            pltpu.SemaphoreType.DMA,
        ],
    )
    def k(x_h, meta_h, val_h, out_h, acc, m0, m1, v0, v1, r0, r1, li0, li1,
          msem, gsem, ssem):
        c = lax.axis_index("c")
        s = lax.axis_index("s")

        _zero_fill(r0, nq=2)
        zbase = s * zrows
        for t in range(z_full):
            pltpu.sync_copy(r0, acc.at[pl.ds(zbase + t * 128, 128)])
        if z_rem:
            pltpu.sync_copy(r0.at[pl.ds(0, z_rem)],
                            acc.at[pl.ds(zbase + z_full * 128, z_rem)])
        plsc.subcore_barrier()

        mrow = s * (3 * tch)
        vrow = s * tch

        def meta_issue(q, mb, vb):
            pltpu.async_copy(meta_h.at[pl.ds(mrow + 3 * q, 3)], mb, msem)
            pltpu.async_copy(val_h.at[pl.ds(vrow + q, 1)], vb, msem)

        def meta_wait(q, mb, vb):
            pltpu.make_async_copy(meta_h.at[pl.ds(mrow + 3 * q, 3)], mb,
                                  msem).wait()
            pltpu.make_async_copy(val_h.at[pl.ds(vrow + q, 1)], vb,
                                  msem).wait()

        def snap_idx(mb, lb):
            for g in range(8):
                lb[0, pl.ds(g * 16, 16)] = mb[2, pl.ds(g * 16, 16)]

        def scatter_issue(lb, rb):
            pltpu.async_copy(rb, acc.at[lb.at[0]], ssem, add=True)

        def scatter_wait(lb, rb):
            pltpu.make_async_copy(rb, acc.at[lb.at[0]], ssem).wait()

        meta_issue(0, m0, v0)
        meta_issue(1, m1, v1)
        meta_wait(0, m0, v0)
        pltpu.async_copy(x_h.at[m0.at[c]], r0, gsem)
        _zero_fill(r1, nq=2)
        snap_idx(m0, li1)
        scatter_issue(li1, r1)

        def body(j, carry):
            g = 2 * j
            meta_wait(g + 1, m1, v1)
            scatter_wait(li1, r1)
            pltpu.async_copy(x_h.at[m1.at[c]], r1, gsem)
            pltpu.make_async_copy(x_h.at[m0.at[c]], r0, gsem).wait()
            snap_idx(m0, li0)
            _scale_rows(r0, v0, nq=2)
            scatter_issue(li0, r0)
            meta_issue(g + 2, m0, v0)
            pltpu.make_async_copy(x_h.at[m1.at[c]], r1, gsem).wait()
            snap_idx(m1, li1)
            _scale_rows(r1, v1, nq=2)
            scatter_issue(li1, r1)
            meta_wait(g + 2, m0, v0)
            scatter_wait(li0, r0)
            pltpu.async_copy(x_h.at[m0.at[c]], r0, gsem)
            meta_issue(g + 3, m1, v1)
            return carry

        lax.fori_loop(0, loop // 2, body, 0)
        scatter_wait(li1, r1)
        pltpu.make_async_copy(x_h.at[m0.at[c]], r0, gsem).wait()
        meta_wait(loop + 1, m1, v1)
        plsc.subcore_barrier()

        obase_l = s * orows
        obase_g = c * out_half + s * orows
        for t in range(o_full):
            pltpu.sync_copy(acc.at[pl.ds(obase_l + t * 128, 128)], r0)
            pltpu.sync_copy(r0, out_h.at[pl.ds(obase_g + t * 128, 128)])
        if o_rem:
            pltpu.sync_copy(acc.at[pl.ds(obase_l + o_full * 128, o_rem)],
                            r0.at[pl.ds(0, o_rem)])
            pltpu.sync_copy(r0.at[pl.ds(0, o_rem)],
                            out_h.at[pl.ds(obase_g + o_full * 128, o_rem)])

    return k(x, meta2d, val2d)


def _combine_body(gg_ref, geB_ref, ge_ref, e1_ref, e2_ref, gi_ref,
                  w_ref, out_ref):
    gge = jnp.dot(gg_ref[...], geB_ref[...],
                  preferred_element_type=jnp.float32)
    hg = (ge_ref[...] + e1_ref[...] + e2_ref[...]) * (1.0 / 3.0)
    gi = gi_ref[...]
    wb = w_ref[...]
    w = wb[:64, :]
    b = wb[64:65, :]
    th = jnp.dot(hg, w[:, 0:1], preferred_element_type=jnp.float32) + b[0, 0]
    tl = jnp.dot(gi, w[:, 1:2], preferred_element_type=jnp.float32) + b[0, 1]
    to = jnp.dot(gge, w[:, 2:3], preferred_element_type=jnp.float32) + b[0, 2]
    out_ref[...] = (jax.nn.sigmoid(th) * hg + jax.nn.sigmoid(tl) * gi
                    + jax.nn.sigmoid(to) * gge)


def _combine_tc(gg_dense, group_emb, e1, e2, gi, wb):
    return pl.pallas_call(
        _combine_body,
        grid=(25,),
        in_specs=[
            pl.BlockSpec((200, 5000), lambda i: (i, 0)),
            pl.BlockSpec((5000, EMB), lambda i: (0, 0)),
            pl.BlockSpec((200, EMB), lambda i: (i, 0)),
            pl.BlockSpec((200, EMB), lambda i: (250 + i, 0)),
            pl.BlockSpec((200, EMB), lambda i: (250 + i, 0)),
            pl.BlockSpec((200, EMB), lambda i: (i, 0)),
            pl.BlockSpec((65, 3), lambda i: (0, 0)),
        ],
        out_specs=pl.BlockSpec((200, EMB), lambda i: (i, 0)),
        out_shape=jax.ShapeDtypeStruct((NG, EMB), jnp.float32),
    )(gg_dense, group_emb, group_emb, e1, e2, gi, wb)


def _batch_gather(user_emb, e1, e2, group_emb, gfin,
                  user_inputs, pos_groups, neg_groups):
    outs = tuple(jax.ShapeDtypeStruct((BATCH, EMB), jnp.float32)
                 for _ in range(6))

    @functools.partial(
        pl.kernel,
        mesh=_mesh,
        compiler_params=pltpu.CompilerParams(use_tc_tiling_on_sc=False),
        out_type=outs,
        scratch_types=[
            pltpu.VMEM((128,), jnp.int32),
            pltpu.VMEM((128, EMB), jnp.float32),
            pltpu.VMEM((128, EMB), jnp.float32),
            pltpu.VMEM((128, EMB), jnp.float32),
            pltpu.SemaphoreType.DMA,
        ],
    )
    def k(ue_h, e1_h, e2_h, ge_h, gf_h, ui_h, pg_h, ng_h,
          o_uemb, o_pos, o_neg, o_uego, o_pego, o_nego,
          idxv, r0, r1, r2, sem):
        c = lax.axis_index("c")
        s = lax.axis_index("s")
        w = s * NC + c
        base = w * 128

        pltpu.sync_copy(ui_h.at[pl.ds(base, 128)], idxv)
        g0 = pltpu.async_copy(ue_h.at[idxv], r0, sem)
        g1 = pltpu.async_copy(e1_h.at[idxv], r1, sem)
        g2 = pltpu.async_copy(e2_h.at[idxv], r2, sem)
        g0.wait(); g1.wait(); g2.wait()
        pltpu.sync_copy(r0, o_uego.at[pl.ds(base, 128)])

        def mean_body(i, carry):
            for q in range(4):
                a = r0[i, pl.ds(q * 16, 16)]
                bq = r1[i, pl.ds(q * 16, 16)]
                cq = r2[i, pl.ds(q * 16, 16)]
                r1[i, pl.ds(q * 16, 16)] = (a + bq + cq) * (1.0 / 3.0)
            return carry

        lax.fori_loop(0, 128, mean_body, 0)
        pltpu.sync_copy(r1, o_uemb.at[pl.ds(base, 128)])

        pltpu.sync_copy(pg_h.at[pl.ds(base, 128)], idxv)
        g0 = pltpu.async_copy(gf_h.at[idxv], r0, sem)
        g1 = pltpu.async_copy(ge_h.at[idxv], r1, sem)
        g0.wait(); g1.wait()
        pltpu.sync_copy(r0, o_pos.at[pl.ds(base, 128)])
        pltpu.sync_copy(r1, o_pego.at[pl.ds(base, 128)])

        pltpu.sync_copy(ng_h.at[pl.ds(base, 128)], idxv)
        g0 = pltpu.async_copy(gf_h.at[idxv], r0, sem)
        g1 = pltpu.async_copy(ge_h.at[idxv], r1, sem)
        g0.wait(); g1.wait()
        pltpu.sync_copy(r0, o_neg.at[pl.ds(base, 128)])
        pltpu.sync_copy(r1, o_nego.at[pl.ds(base, 128)])

    return k(user_emb, e1, e2, group_emb, gfin,
             user_inputs, pos_groups, neg_groups)


def _pack_chunks(arrs, n_tiles, loop_chunks, tch, pads):
    out = []
    for a, padv in zip(arrs, pads):
        n = n_tiles * loop_chunks * 128 - a.shape[0]
        ap = jnp.concatenate([a, jnp.full((n,), padv, a.dtype)])
        ap = ap.reshape(n_tiles, loop_chunks, 128)
        tail = jnp.full((n_tiles, tch - loop_chunks, 128), padv, a.dtype)
        out.append(jnp.concatenate([ap, tail], axis=1))
    return out


def _prep_hg_edges(row, col, val):
    c1 = col + NHG_PAD
    cp, c1p, rp, vp = _pack_chunks(
        [col, c1, row, val], NS, HG_LOOP, HG_TCH, [0, NHG_PAD, NHG_PAD, 0.0])
    meta = jnp.stack([cp, c1p, rp], axis=2)
    return meta.reshape(NS * HG_TCH * 3, 128), vp.reshape(NS * HG_TCH, 128)


def _prep_gi_edges(row, col, val):
    c1 = col + NGI
    li = jnp.where(row < NG, row, NG)
    cp, c1p, lp, vp = _pack_chunks(
        [col, c1, li, val], NS, GI_LOOP, GI_TCH, [0, NGI, NG, 0.0])
    meta = jnp.stack([cp, c1p, lp], axis=2)
    return meta.reshape(NS * GI_TCH * 3, 128), vp.reshape(NS * GI_TCH, 128)


def kernel(user_emb, item_emb, group_emb, hg_vals, gi_vals, gg_dense,
           hyper_W, hyper_b, light_W, light_b, over_W, over_b,
           hg_row, hg_col, gi_row, gi_col,
           user_inputs, pos_groups, neg_groups):
    x0 = jnp.concatenate([
        user_emb, group_emb,
        jnp.zeros((NHG_PAD - NHG, EMB), jnp.float32)], axis=0)
    x0cs = jnp.concatenate([x0[:, :32], x0[:, 32:]], axis=0)
    hmeta, hval = _prep_hg_edges(hg_row, hg_col, hg_vals)
    e1cs = _spmm_colsplit(x0cs, hmeta, hval, NHG_PAD, ACC_CS,
                          HG_LOOP, HG_TCH)
    e2cs = _spmm_colsplit(e1cs, hmeta, hval, NHG_PAD, ACC_CS,
                          HG_LOOP, HG_TCH)
    e1 = jnp.concatenate([e1cs[:NHG_PAD], e1cs[NHG_PAD:]], axis=1)
    e2 = jnp.concatenate([e2cs[:NHG_PAD], e2cs[NHG_PAD:]], axis=1)

    xg = jnp.concatenate([group_emb, item_emb], axis=0)
    xgcs = jnp.concatenate([xg[:, :32], xg[:, 32:]], axis=0)
    gmeta, gval = _prep_gi_edges(gi_row, gi_col, gi_vals)
    gics = _spmm_colsplit(xgcs, gmeta, gval, GI_ACC, GI_ACC,
                          GI_LOOP, GI_TCH)
    gi = jnp.concatenate([gics[:GI_ACC], gics[GI_ACC:]], axis=1)[:NG]

    wb = jnp.concatenate([
        jnp.concatenate([hyper_W, light_W, over_W], axis=1),
        jnp.stack([hyper_b[0], light_b[0], over_b[0]])[None, :],
    ], axis=0)
    gfin = _combine_tc(gg_dense, group_emb, e1, e2, gi, wb)

    return _batch_gather(user_emb, e1, e2, group_emb, gfin,
                         user_inputs, pos_groups, neg_groups)


# speedup vs baseline: 1.0393x; 1.0393x over previous
import functools

import jax
import jax.numpy as jnp
from jax import lax
from jax.experimental import pallas as pl
from jax.experimental.pallas import tpu as pltpu
from jax.experimental.pallas import tpu_sc as plsc

NU = 50000
NG = 5000
NI = 20000
EMB = 64
NHG = NU + NG
NGI = NG + NI
EHG = 880000
EGI = 400000
BATCH = 4096

NC = 2
NS = 16

NHG_PAD = 55296
ACC_CS = 55312
HG_LOOP = 432
HG_TCH = 434

ACC_GI = 5120
GI_LOOP = 98
GI_TCH = 100

_mesh = plsc.VectorSubcoreMesh(core_axis_name="c", subcore_axis_name="s")


def _zero_fill(zbuf, nq=4):
    zv = jnp.zeros((16,), jnp.float32)
    for i in range(128):
        for q in range(nq):
            zbuf[i, pl.ds(q * 16, 16)] = zv


def _scale_rows(rows_v, vb, nq=4):
    def scale_body(g, carry2):
        e0 = g * 16
        vv = vb[0, pl.ds(g * 16, 16)]
        for t in range(16):
            e = e0 + t
            vs = vv.at[jnp.full((16,), t, jnp.int32)].get(
                mode="promise_in_bounds")
            for q in range(nq):
                sl = rows_v[e, pl.ds(q * 16, 16)]
                rows_v[e, pl.ds(q * 16, 16)] = sl * vs
        return carry2

    lax.fori_loop(0, 8, scale_body, 0)


def _spmm_colsplit(x, meta2d, val2d, out_half, acc_rows, loop, tch):
    zrows = acc_rows // NS
    z_full, z_rem = divmod(zrows, 128)
    orows = out_half // NS
    o_full, o_rem = divmod(orows, 128)

    @functools.partial(
        pl.kernel,
        mesh=_mesh,
        compiler_params=pltpu.CompilerParams(use_tc_tiling_on_sc=False),
        out_type=jax.ShapeDtypeStruct((2 * out_half, 32), jnp.float32),
        scratch_types=[
            pltpu.VMEM_SHARED((acc_rows, 32), jnp.float32),
            pltpu.VMEM((3, 128), jnp.int32),
            pltpu.VMEM((3, 128), jnp.int32),
            pltpu.VMEM((1, 128), jnp.float32),
            pltpu.VMEM((1, 128), jnp.float32),
            pltpu.VMEM((128, 32), jnp.float32),
            pltpu.VMEM((128, 32), jnp.float32),
            pltpu.VMEM((1, 128), jnp.int32),
            pltpu.VMEM((1, 128), jnp.int32),
            pltpu.SemaphoreType.DMA,
            pltpu.SemaphoreType.DMA,
            pltpu.SemaphoreType.DMA,
        ],
    )
    def k(x_h, meta_h, val_h, out_h, acc, m0, m1, v0, v1, r0, r1, li0, li1,
          msem, gsem, ssem):
        c = lax.axis_index("c")
        s = lax.axis_index("s")

        _zero_fill(r0, nq=2)
        zbase = s * zrows
        for t in range(z_full):
            pltpu.sync_copy(r0, acc.at[pl.ds(zbase + t * 128, 128)])
        if z_rem:
            pltpu.sync_copy(r0.at[pl.ds(0, z_rem)],
                            acc.at[pl.ds(zbase + z_full * 128, z_rem)])
        plsc.subcore_barrier()

        mrow = s * (3 * tch)
        vrow = s * tch

        def meta_issue(q, mb, vb):
            pltpu.async_copy(meta_h.at[pl.ds(mrow + 3 * q, 3)], mb, msem)
            pltpu.async_copy(val_h.at[pl.ds(vrow + q, 1)], vb, msem)

        def meta_wait(q, mb, vb):
            pltpu.make_async_copy(meta_h.at[pl.ds(mrow + 3 * q, 3)], mb,
                                  msem).wait()
            pltpu.make_async_copy(val_h.at[pl.ds(vrow + q, 1)], vb,
                                  msem).wait()

        def snap_idx(mb, lb):
            for g in range(8):
                lb[0, pl.ds(g * 16, 16)] = mb[2, pl.ds(g * 16, 16)]

        def scatter_issue(lb, rb):
            pltpu.async_copy(rb, acc.at[lb.at[0]], ssem, add=True)

        def scatter_wait(lb, rb):
            pltpu.make_async_copy(rb, acc.at[lb.at[0]], ssem).wait()

        meta_issue(0, m0, v0)
        meta_issue(1, m1, v1)
        meta_wait(0, m0, v0)
        pltpu.async_copy(x_h.at[m0.at[c]], r0, gsem)
        _zero_fill(r1, nq=2)
        snap_idx(m0, li1)
        scatter_issue(li1, r1)

        def body(j, carry):
            g = 2 * j
            meta_wait(g + 1, m1, v1)
            scatter_wait(li1, r1)
            pltpu.async_copy(x_h.at[m1.at[c]], r1, gsem)
            pltpu.make_async_copy(x_h.at[m0.at[c]], r0, gsem).wait()
            snap_idx(m0, li0)
            _scale_rows(r0, v0, nq=2)
            scatter_issue(li0, r0)
            meta_issue(g + 2, m0, v0)
            pltpu.make_async_copy(x_h.at[m1.at[c]], r1, gsem).wait()
            snap_idx(m1, li1)
            _scale_rows(r1, v1, nq=2)
            scatter_issue(li1, r1)
            meta_wait(g + 2, m0, v0)
            scatter_wait(li0, r0)
            pltpu.async_copy(x_h.at[m0.at[c]], r0, gsem)
            meta_issue(g + 3, m1, v1)
            return carry

        lax.fori_loop(0, loop // 2, body, 0)
        scatter_wait(li1, r1)
        pltpu.make_async_copy(x_h.at[m0.at[c]], r0, gsem).wait()
        meta_wait(loop + 1, m1, v1)
        plsc.subcore_barrier()

        obase_l = s * orows
        obase_g = c * out_half + s * orows
        for t in range(o_full):
            pltpu.sync_copy(acc.at[pl.ds(obase_l + t * 128, 128)], r0)
            pltpu.sync_copy(r0, out_h.at[pl.ds(obase_g + t * 128, 128)])
        if o_rem:
            pltpu.sync_copy(acc.at[pl.ds(obase_l + o_full * 128, o_rem)],
                            r0.at[pl.ds(0, o_rem)])
            pltpu.sync_copy(r0.at[pl.ds(0, o_rem)],
                            out_h.at[pl.ds(obase_g + o_full * 128, o_rem)])

    return k(x, meta2d, val2d)


def _spmm_gi(x, meta2d, val2d):
    @functools.partial(
        pl.kernel,
        mesh=_mesh,
        compiler_params=pltpu.CompilerParams(use_tc_tiling_on_sc=False),
        out_type=jax.ShapeDtypeStruct((2 * ACC_GI, EMB), jnp.float32),
        scratch_types=[
            pltpu.VMEM_SHARED((ACC_GI, EMB), jnp.float32),
            pltpu.VMEM((2, 128), jnp.int32),
            pltpu.VMEM((2, 128), jnp.int32),
            pltpu.VMEM((1, 128), jnp.float32),
            pltpu.VMEM((1, 128), jnp.float32),
            pltpu.VMEM((128, EMB), jnp.float32),
            pltpu.VMEM((128, EMB), jnp.float32),
            pltpu.VMEM((1, 128), jnp.int32),
            pltpu.VMEM((1, 128), jnp.int32),
            pltpu.SemaphoreType.DMA,
            pltpu.SemaphoreType.DMA,
            pltpu.SemaphoreType.DMA,
        ],
    )
    def k(x_h, meta_h, val_h, out_h, acc, m0, m1, v0, v1, r0, r1, li0, li1,
          msem, gsem, ssem):
        c = lax.axis_index("c")
        s = lax.axis_index("s")
        w = s * NC + c

        _zero_fill(r0)
        zbase = s * 320
        for t in range(2):
            pltpu.sync_copy(r0, acc.at[pl.ds(zbase + t * 128, 128)])
        pltpu.sync_copy(r0.at[pl.ds(0, 64)], acc.at[pl.ds(zbase + 256, 64)])
        plsc.subcore_barrier()

        mrow = w * (2 * GI_TCH)
        vrow = w * GI_TCH

        def meta_issue(q, mb, vb):
            pltpu.async_copy(meta_h.at[pl.ds(mrow + 2 * q, 2)], mb, msem)
            pltpu.async_copy(val_h.at[pl.ds(vrow + q, 1)], vb, msem)

        def meta_wait(q, mb, vb):
            pltpu.make_async_copy(meta_h.at[pl.ds(mrow + 2 * q, 2)], mb,
                                  msem).wait()
            pltpu.make_async_copy(val_h.at[pl.ds(vrow + q, 1)], vb,
                                  msem).wait()

        def snap_idx(mb, lb):
            for g in range(8):
                lb[0, pl.ds(g * 16, 16)] = mb[1, pl.ds(g * 16, 16)]

        def scatter_issue(lb, rb):
            pltpu.async_copy(rb, acc.at[lb.at[0]], ssem, add=True)

        def scatter_wait(lb, rb):
            pltpu.make_async_copy(rb, acc.at[lb.at[0]], ssem).wait()

        meta_issue(0, m0, v0)
        meta_issue(1, m1, v1)
        meta_wait(0, m0, v0)
        pltpu.async_copy(x_h.at[m0.at[0]], r0, gsem)
        _zero_fill(r1)
        snap_idx(m0, li1)
        scatter_issue(li1, r1)

        def body(j, carry):
            g = 2 * j
            meta_wait(g + 1, m1, v1)
            scatter_wait(li1, r1)
            pltpu.async_copy(x_h.at[m1.at[0]], r1, gsem)
            pltpu.make_async_copy(x_h.at[m0.at[0]], r0, gsem).wait()
            snap_idx(m0, li0)
            _scale_rows(r0, v0)
            scatter_issue(li0, r0)
            meta_issue(g + 2, m0, v0)
            pltpu.make_async_copy(x_h.at[m1.at[0]], r1, gsem).wait()
            snap_idx(m1, li1)
            _scale_rows(r1, v1)
            scatter_issue(li1, r1)
            meta_wait(g + 2, m0, v0)
            scatter_wait(li0, r0)
            pltpu.async_copy(x_h.at[m0.at[0]], r0, gsem)
            meta_issue(g + 3, m1, v1)
            return carry

        lax.fori_loop(0, GI_LOOP // 2, body, 0)
        scatter_wait(li1, r1)
        pltpu.make_async_copy(x_h.at[m0.at[0]], r0, gsem).wait()
        meta_wait(GI_LOOP + 1, m1, v1)
        plsc.subcore_barrier()

        obase_l = s * 320
        obase_g = c * ACC_GI + s * 320
        for t in range(2):
            pltpu.sync_copy(acc.at[pl.ds(obase_l + t * 128, 128)], r0)
            pltpu.sync_copy(r0, out_h.at[pl.ds(obase_g + t * 128, 128)])
        pltpu.sync_copy(acc.at[pl.ds(obase_l + 256, 64)],
                        r0.at[pl.ds(0, 64)])
        pltpu.sync_copy(r0.at[pl.ds(0, 64)],
                        out_h.at[pl.ds(obase_g + 256, 64)])

    return k(x, meta2d, val2d)


def _combine_body(gg_ref, geB_ref, ge_ref, e1_ref, e2_ref, gi0_ref, gi1_ref,
                  w_ref, out_ref):
    gge = jnp.dot(gg_ref[...], geB_ref[...],
                  preferred_element_type=jnp.float32)
    hg = (ge_ref[...] + e1_ref[...] + e2_ref[...]) * (1.0 / 3.0)
    gi = gi0_ref[...] + gi1_ref[...]
    wb = w_ref[...]
    w = wb[:64, :]
    b = wb[64:65, :]
    th = jnp.dot(hg, w[:, 0:1], preferred_element_type=jnp.float32) + b[0, 0]
    tl = jnp.dot(gi, w[:, 1:2], preferred_element_type=jnp.float32) + b[0, 1]
    to = jnp.dot(gge, w[:, 2:3], preferred_element_type=jnp.float32) + b[0, 2]
    out_ref[...] = (jax.nn.sigmoid(th) * hg + jax.nn.sigmoid(tl) * gi
                    + jax.nn.sigmoid(to) * gge)


def _combine_tc(gg_dense, group_emb, e1, e2, gi0, gi1, wb):
    return pl.pallas_call(
        _combine_body,
        grid=(25,),
        in_specs=[
            pl.BlockSpec((200, 5000), lambda i: (i, 0)),
            pl.BlockSpec((5000, EMB), lambda i: (0, 0)),
            pl.BlockSpec((200, EMB), lambda i: (i, 0)),
            pl.BlockSpec((200, EMB), lambda i: (250 + i, 0)),
            pl.BlockSpec((200, EMB), lambda i: (250 + i, 0)),
            pl.BlockSpec((200, EMB), lambda i: (i, 0)),
            pl.BlockSpec((200, EMB), lambda i: (i, 0)),
            pl.BlockSpec((65, 3), lambda i: (0, 0)),
        ],
        out_specs=pl.BlockSpec((200, EMB), lambda i: (i, 0)),
        out_shape=jax.ShapeDtypeStruct((NG, EMB), jnp.float32),
    )(gg_dense, group_emb, group_emb, e1, e2, gi0, gi1, wb)


def _batch_gather(user_emb, e1, e2, group_emb, gfin,
                  user_inputs, pos_groups, neg_groups):
    outs = tuple(jax.ShapeDtypeStruct((BATCH, EMB), jnp.float32)
                 for _ in range(6))

    @functools.partial(
        pl.kernel,
        mesh=_mesh,
        compiler_params=pltpu.CompilerParams(use_tc_tiling_on_sc=False),
        out_type=outs,
        scratch_types=[
            pltpu.VMEM((128,), jnp.int32),
            pltpu.VMEM((128, EMB), jnp.float32),
            pltpu.VMEM((128, EMB), jnp.float32),
            pltpu.VMEM((128, EMB), jnp.float32),
            pltpu.SemaphoreType.DMA,
        ],
    )
    def k(ue_h, e1_h, e2_h, ge_h, gf_h, ui_h, pg_h, ng_h,
          o_uemb, o_pos, o_neg, o_uego, o_pego, o_nego,
          idxv, r0, r1, r2, sem):
        c = lax.axis_index("c")
        s = lax.axis_index("s")
        w = s * NC + c
        base = w * 128

        pltpu.sync_copy(ui_h.at[pl.ds(base, 128)], idxv)
        g0 = pltpu.async_copy(ue_h.at[idxv], r0, sem)
        g1 = pltpu.async_copy(e1_h.at[idxv], r1, sem)
        g2 = pltpu.async_copy(e2_h.at[idxv], r2, sem)
        g0.wait(); g1.wait(); g2.wait()
        pltpu.sync_copy(r0, o_uego.at[pl.ds(base, 128)])

        def mean_body(i, carry):
            for q in range(4):
                a = r0[i, pl.ds(q * 16, 16)]
                bq = r1[i, pl.ds(q * 16, 16)]
                cq = r2[i, pl.ds(q * 16, 16)]
                r1[i, pl.ds(q * 16, 16)] = (a + bq + cq) * (1.0 / 3.0)
            return carry

        lax.fori_loop(0, 128, mean_body, 0)
        pltpu.sync_copy(r1, o_uemb.at[pl.ds(base, 128)])

        pltpu.sync_copy(pg_h.at[pl.ds(base, 128)], idxv)
        g0 = pltpu.async_copy(gf_h.at[idxv], r0, sem)
        g1 = pltpu.async_copy(ge_h.at[idxv], r1, sem)
        g0.wait(); g1.wait()
        pltpu.sync_copy(r0, o_pos.at[pl.ds(base, 128)])
        pltpu.sync_copy(r1, o_pego.at[pl.ds(base, 128)])

        pltpu.sync_copy(ng_h.at[pl.ds(base, 128)], idxv)
        g0 = pltpu.async_copy(gf_h.at[idxv], r0, sem)
        g1 = pltpu.async_copy(ge_h.at[idxv], r1, sem)
        g0.wait(); g1.wait()
        pltpu.sync_copy(r0, o_neg.at[pl.ds(base, 128)])
        pltpu.sync_copy(r1, o_nego.at[pl.ds(base, 128)])

    return k(user_emb, e1, e2, group_emb, gfin,
             user_inputs, pos_groups, neg_groups)


def _pack_chunks(arrs, n_tiles, loop_chunks, tch, pads):
    out = []
    for a, padv in zip(arrs, pads):
        n = n_tiles * loop_chunks * 128 - a.shape[0]
        ap = jnp.concatenate([a, jnp.full((n,), padv, a.dtype)])
        ap = ap.reshape(n_tiles, loop_chunks, 128)
        tail = jnp.full((n_tiles, tch - loop_chunks, 128), padv, a.dtype)
        out.append(jnp.concatenate([ap, tail], axis=1))
    return out


def _prep_hg_edges(row, col, val):
    c1 = col + NHG_PAD
    cp, c1p, rp, vp = _pack_chunks(
        [col, c1, row, val], NS, HG_LOOP, HG_TCH, [0, NHG_PAD, NHG_PAD, 0.0])
    meta = jnp.stack([cp, c1p, rp], axis=2)
    return meta.reshape(NS * HG_TCH * 3, 128), vp.reshape(NS * HG_TCH, 128)


def _prep_gi_edges(row, col, val):
    li = jnp.where(row < NG, row, NG)
    cp, lp, vp = _pack_chunks(
        [col, li, val], NC * NS, GI_LOOP, GI_TCH, [0, NG, 0.0])
    meta = jnp.stack([cp, lp], axis=2)
    return (meta.reshape(NC * NS * GI_TCH * 2, 128),
            vp.reshape(NC * NS * GI_TCH, 128))


def kernel(user_emb, item_emb, group_emb, hg_vals, gi_vals, gg_dense,
           hyper_W, hyper_b, light_W, light_b, over_W, over_b,
           hg_row, hg_col, gi_row, gi_col,
           user_inputs, pos_groups, neg_groups):
    x0 = jnp.concatenate([
        user_emb, group_emb,
        jnp.zeros((NHG_PAD - NHG, EMB), jnp.float32)], axis=0)
    x0cs = jnp.concatenate([x0[:, :32], x0[:, 32:]], axis=0)
    hmeta, hval = _prep_hg_edges(hg_row, hg_col, hg_vals)
    e1cs = _spmm_colsplit(x0cs, hmeta, hval, NHG_PAD, ACC_CS,
                          HG_LOOP, HG_TCH)
    e2cs = _spmm_colsplit(e1cs, hmeta, hval, NHG_PAD, ACC_CS,
                          HG_LOOP, HG_TCH)
    e1 = jnp.concatenate([e1cs[:NHG_PAD], e1cs[NHG_PAD:]], axis=1)
    e2 = jnp.concatenate([e2cs[:NHG_PAD], e2cs[NHG_PAD:]], axis=1)

    xg = jnp.concatenate([group_emb, item_emb], axis=0)
    gmeta, gval = _prep_gi_edges(gi_row, gi_col, gi_vals)
    gip = _spmm_gi(xg, gmeta, gval)

    wb = jnp.concatenate([
        jnp.concatenate([hyper_W, light_W, over_W], axis=1),
        jnp.stack([hyper_b[0], light_b[0], over_b[0]])[None, :],
    ], axis=0)
    gfin = _combine_tc(gg_dense, group_emb, e1, e2,
                       gip[:ACC_GI], gip[ACC_GI:], wb)

    return _batch_gather(user_emb, e1, e2, group_emb, gfin,
                         user_inputs, pos_groups, neg_groups)


# speedup vs baseline: 1.0511x; 1.0114x over previous
import functools

import jax
import jax.numpy as jnp
from jax import lax
from jax.experimental import pallas as pl
from jax.experimental.pallas import tpu as pltpu
from jax.experimental.pallas import tpu_sc as plsc

NU = 50000
NG = 5000
NI = 20000
EMB = 64
NHG = NU + NG
NGI = NG + NI
EHG = 880000
EGI = 400000
BATCH = 4096

NC = 2
NS = 16

NHG_PAD = 55296
ACC_CS = 55312
HG_LOOP = 432
HG_TCH = 434

ACC_GI = 5120
GI_LOOP = 98
GI_TCH = 100

_mesh = plsc.VectorSubcoreMesh(core_axis_name="c", subcore_axis_name="s")


def _zero_fill(zbuf, nq=4):
    zv = jnp.zeros((16,), jnp.float32)
    for i in range(128):
        for q in range(nq):
            zbuf[i, pl.ds(q * 16, 16)] = zv


def _scale_rows(rows_v, vb, nq=4):
    def scale_body(g, carry2):
        e0 = g * 16
        vv = vb[0, pl.ds(g * 16, 16)]
        for t in range(16):
            e = e0 + t
            vs = vv.at[jnp.full((16,), t, jnp.int32)].get(
                mode="promise_in_bounds")
            for q in range(nq):
                sl = rows_v[e, pl.ds(q * 16, 16)]
                rows_v[e, pl.ds(q * 16, 16)] = sl * vs
        return carry2

    lax.fori_loop(0, 8, scale_body, 0)


def _spmm_colsplit(x, meta2d, val2d, out_half, acc_rows, loop, tch):
    zrows = acc_rows // NS
    z_full, z_rem = divmod(zrows, 128)
    orows = out_half // NS
    o_full, o_rem = divmod(orows, 128)

    @functools.partial(
        pl.kernel,
        mesh=_mesh,
        compiler_params=pltpu.CompilerParams(use_tc_tiling_on_sc=False),
        out_type=jax.ShapeDtypeStruct((2 * out_half, 32), jnp.float32),
        scratch_types=[
            pltpu.VMEM_SHARED((acc_rows, 32), jnp.float32),
            pltpu.VMEM((3, 128), jnp.int32),
            pltpu.VMEM((3, 128), jnp.int32),
            pltpu.VMEM((1, 128), jnp.float32),
            pltpu.VMEM((1, 128), jnp.float32),
            pltpu.VMEM((128, 32), jnp.float32),
            pltpu.VMEM((128, 32), jnp.float32),
            pltpu.VMEM((1, 128), jnp.int32),
            pltpu.VMEM((1, 128), jnp.int32),
            pltpu.SemaphoreType.DMA,
            pltpu.SemaphoreType.DMA,
            pltpu.SemaphoreType.DMA,
        ],
    )
    def k(x_h, meta_h, val_h, out_h, acc, m0, m1, v0, v1, r0, r1, li0, li1,
          msem, gsem, ssem):
        c = lax.axis_index("c")
        s = lax.axis_index("s")

        _zero_fill(r0, nq=2)
        zbase = s * zrows
        for t in range(z_full):
            pltpu.sync_copy(r0, acc.at[pl.ds(zbase + t * 128, 128)])
        if z_rem:
            pltpu.sync_copy(r0.at[pl.ds(0, z_rem)],
                            acc.at[pl.ds(zbase + z_full * 128, z_rem)])
        plsc.subcore_barrier()

        mrow = s * (3 * tch)
        vrow = s * tch

        def meta_issue(q, mb, vb):
            pltpu.async_copy(meta_h.at[pl.ds(mrow + 3 * q, 3)], mb, msem)
            pltpu.async_copy(val_h.at[pl.ds(vrow + q, 1)], vb, msem)

        def meta_wait(q, mb, vb):
            pltpu.make_async_copy(meta_h.at[pl.ds(mrow + 3 * q, 3)], mb,
                                  msem).wait()
            pltpu.make_async_copy(val_h.at[pl.ds(vrow + q, 1)], vb,
                                  msem).wait()

        def snap_idx(mb, lb):
            for g in range(8):
                lb[0, pl.ds(g * 16, 16)] = mb[2, pl.ds(g * 16, 16)]

        def scatter_issue(lb, rb):
            pltpu.async_copy(rb, acc.at[lb.at[0]], ssem, add=True)

        def scatter_wait(lb, rb):
            pltpu.make_async_copy(rb, acc.at[lb.at[0]], ssem).wait()

        meta_issue(0, m0, v0)
        meta_issue(1, m1, v1)
        meta_wait(0, m0, v0)
        pltpu.async_copy(x_h.at[m0.at[c]], r0, gsem)
        _zero_fill(r1, nq=2)
        snap_idx(m0, li1)
        scatter_issue(li1, r1)

        def body(j, carry):
            g = 2 * j
            meta_wait(g + 1, m1, v1)
            scatter_wait(li1, r1)
            pltpu.async_copy(x_h.at[m1.at[c]], r1, gsem)
            pltpu.make_async_copy(x_h.at[m0.at[c]], r0, gsem).wait()
            snap_idx(m0, li0)
            _scale_rows(r0, v0, nq=2)
            scatter_issue(li0, r0)
            meta_issue(g + 2, m0, v0)
            pltpu.make_async_copy(x_h.at[m1.at[c]], r1, gsem).wait()
            snap_idx(m1, li1)
            _scale_rows(r1, v1, nq=2)
            scatter_issue(li1, r1)
            meta_wait(g + 2, m0, v0)
            scatter_wait(li0, r0)
            pltpu.async_copy(x_h.at[m0.at[c]], r0, gsem)
            meta_issue(g + 3, m1, v1)
            return carry

        lax.fori_loop(0, loop // 2, body, 0)
        scatter_wait(li1, r1)
        pltpu.make_async_copy(x_h.at[m0.at[c]], r0, gsem).wait()
        meta_wait(loop + 1, m1, v1)
        plsc.subcore_barrier()

        obase_l = s * orows
        obase_g = c * out_half + s * orows
        for t in range(o_full):
            pltpu.sync_copy(acc.at[pl.ds(obase_l + t * 128, 128)], r0)
            pltpu.sync_copy(r0, out_h.at[pl.ds(obase_g + t * 128, 128)])
        if o_rem:
            pltpu.sync_copy(acc.at[pl.ds(obase_l + o_full * 128, o_rem)],
                            r0.at[pl.ds(0, o_rem)])
            pltpu.sync_copy(r0.at[pl.ds(0, o_rem)],
                            out_h.at[pl.ds(obase_g + o_full * 128, o_rem)])

    return k(x, meta2d, val2d)


def _spmm_gi(x, meta2d, val2d):
    @functools.partial(
        pl.kernel,
        mesh=_mesh,
        compiler_params=pltpu.CompilerParams(use_tc_tiling_on_sc=False),
        out_type=jax.ShapeDtypeStruct((2 * ACC_GI, EMB), jnp.float32),
        scratch_types=[
            pltpu.VMEM_SHARED((ACC_GI, EMB), jnp.float32),
            pltpu.VMEM((2, 128), jnp.int32),
            pltpu.VMEM((2, 128), jnp.int32),
            pltpu.VMEM((1, 128), jnp.float32),
            pltpu.VMEM((1, 128), jnp.float32),
            pltpu.VMEM((128, EMB), jnp.float32),
            pltpu.VMEM((128, EMB), jnp.float32),
            pltpu.VMEM((1, 128), jnp.int32),
            pltpu.VMEM((1, 128), jnp.int32),
            pltpu.SemaphoreType.DMA,
            pltpu.SemaphoreType.DMA,
            pltpu.SemaphoreType.DMA,
        ],
    )
    def k(x_h, meta_h, val_h, out_h, acc, m0, m1, v0, v1, r0, r1, li0, li1,
          msem, gsem, ssem):
        c = lax.axis_index("c")
        s = lax.axis_index("s")
        w = s * NC + c

        _zero_fill(r0)
        zbase = s * 320
        for t in range(2):
            pltpu.sync_copy(r0, acc.at[pl.ds(zbase + t * 128, 128)])
        pltpu.sync_copy(r0.at[pl.ds(0, 64)], acc.at[pl.ds(zbase + 256, 64)])
        plsc.subcore_barrier()

        mrow = w * (2 * GI_TCH)
        vrow = w * GI_TCH

        def meta_issue(q, mb, vb):
            pltpu.async_copy(meta_h.at[pl.ds(mrow + 2 * q, 2)], mb, msem)
            pltpu.async_copy(val_h.at[pl.ds(vrow + q, 1)], vb, msem)

        def meta_wait(q, mb, vb):
            pltpu.make_async_copy(meta_h.at[pl.ds(mrow + 2 * q, 2)], mb,
                                  msem).wait()
            pltpu.make_async_copy(val_h.at[pl.ds(vrow + q, 1)], vb,
                                  msem).wait()

        def snap_idx(mb, lb):
            for g in range(8):
                lb[0, pl.ds(g * 16, 16)] = mb[1, pl.ds(g * 16, 16)]

        def scatter_issue(lb, rb):
            pltpu.async_copy(rb, acc.at[lb.at[0]], ssem, add=True)

        def scatter_wait(lb, rb):
            pltpu.make_async_copy(rb, acc.at[lb.at[0]], ssem).wait()

        meta_issue(0, m0, v0)
        meta_issue(1, m1, v1)
        meta_wait(0, m0, v0)
        pltpu.async_copy(x_h.at[m0.at[0]], r0, gsem)
        _zero_fill(r1)
        snap_idx(m0, li1)
        scatter_issue(li1, r1)

        def body(j, carry):
            g = 2 * j
            meta_wait(g + 1, m1, v1)
            scatter_wait(li1, r1)
            pltpu.async_copy(x_h.at[m1.at[0]], r1, gsem)
            pltpu.make_async_copy(x_h.at[m0.at[0]], r0, gsem).wait()
            snap_idx(m0, li0)
            _scale_rows(r0, v0)
            scatter_issue(li0, r0)
            meta_issue(g + 2, m0, v0)
            pltpu.make_async_copy(x_h.at[m1.at[0]], r1, gsem).wait()
            snap_idx(m1, li1)
            _scale_rows(r1, v1)
            scatter_issue(li1, r1)
            meta_wait(g + 2, m0, v0)
            scatter_wait(li0, r0)
            pltpu.async_copy(x_h.at[m0.at[0]], r0, gsem)
            meta_issue(g + 3, m1, v1)
            return carry

        lax.fori_loop(0, GI_LOOP // 2, body, 0)
        scatter_wait(li1, r1)
        pltpu.make_async_copy(x_h.at[m0.at[0]], r0, gsem).wait()
        meta_wait(GI_LOOP + 1, m1, v1)
        plsc.subcore_barrier()

        obase_l = s * 320
        obase_g = c * ACC_GI + s * 320
        for t in range(2):
            pltpu.sync_copy(acc.at[pl.ds(obase_l + t * 128, 128)], r0)
            pltpu.sync_copy(r0, out_h.at[pl.ds(obase_g + t * 128, 128)])
        pltpu.sync_copy(acc.at[pl.ds(obase_l + 256, 64)],
                        r0.at[pl.ds(0, 64)])
        pltpu.sync_copy(r0.at[pl.ds(0, 64)],
                        out_h.at[pl.ds(obase_g + 256, 64)])

    return k(x, meta2d, val2d)


def _ggmm_body(gg_ref, ge_ref, out_ref):
    out_ref[...] = jnp.dot(gg_ref[...], ge_ref[...],
                           preferred_element_type=jnp.float32)


def _ggmm_tc(gg_dense, group_emb):
    return pl.pallas_call(
        _ggmm_body,
        grid=(25,),
        in_specs=[
            pl.BlockSpec((200, 5000), lambda i: (i, 0)),
            pl.BlockSpec((5000, EMB), lambda i: (0, 0)),
        ],
        out_specs=pl.BlockSpec((200, EMB), lambda i: (i, 0)),
        out_shape=jax.ShapeDtypeStruct((NG, EMB), jnp.float32),
    )(gg_dense, group_emb)


def _combine_body(gge_ref, ge_ref, e1_ref, e2_ref, gi0_ref, gi1_ref,
                  w_ref, out_ref):
    gge = gge_ref[...]
    hg = (ge_ref[...] + e1_ref[...] + e2_ref[...]) * (1.0 / 3.0)
    gi = gi0_ref[...] + gi1_ref[...]
    wb = w_ref[...]
    w = wb[:64, :]
    b = wb[64:65, :]
    th = jnp.dot(hg, w[:, 0:1], preferred_element_type=jnp.float32) + b[0, 0]
    tl = jnp.dot(gi, w[:, 1:2], preferred_element_type=jnp.float32) + b[0, 1]
    to = jnp.dot(gge, w[:, 2:3], preferred_element_type=jnp.float32) + b[0, 2]
    out_ref[...] = (jax.nn.sigmoid(th) * hg + jax.nn.sigmoid(tl) * gi
                    + jax.nn.sigmoid(to) * gge)


def _combine_tc(gge, group_emb, e1, e2, gi0, gi1, wb):
    return pl.pallas_call(
        _combine_body,
        grid=(25,),
        in_specs=[
            pl.BlockSpec((200, EMB), lambda i: (i, 0)),
            pl.BlockSpec((200, EMB), lambda i: (i, 0)),
            pl.BlockSpec((200, EMB), lambda i: (250 + i, 0)),
            pl.BlockSpec((200, EMB), lambda i: (250 + i, 0)),
            pl.BlockSpec((200, EMB), lambda i: (i, 0)),
            pl.BlockSpec((200, EMB), lambda i: (i, 0)),
            pl.BlockSpec((65, 3), lambda i: (0, 0)),
        ],
        out_specs=pl.BlockSpec((200, EMB), lambda i: (i, 0)),
        out_shape=jax.ShapeDtypeStruct((NG, EMB), jnp.float32),
    )(gge, group_emb, e1, e2, gi0, gi1, wb)


def _batch_gather(user_emb, e1, e2, group_emb, gfin,
                  user_inputs, pos_groups, neg_groups):
    outs = tuple(jax.ShapeDtypeStruct((BATCH, EMB), jnp.float32)
                 for _ in range(6))

    @functools.partial(
        pl.kernel,
        mesh=_mesh,
        compiler_params=pltpu.CompilerParams(use_tc_tiling_on_sc=False),
        out_type=outs,
        scratch_types=[
            pltpu.VMEM((128,), jnp.int32),
            pltpu.VMEM((128, EMB), jnp.float32),
            pltpu.VMEM((128, EMB), jnp.float32),
            pltpu.VMEM((128, EMB), jnp.float32),
            pltpu.SemaphoreType.DMA,
        ],
    )
    def k(ue_h, e1_h, e2_h, ge_h, gf_h, ui_h, pg_h, ng_h,
          o_uemb, o_pos, o_neg, o_uego, o_pego, o_nego,
          idxv, r0, r1, r2, sem):
        c = lax.axis_index("c")
        s = lax.axis_index("s")
        w = s * NC + c
        base = w * 128

        pltpu.sync_copy(ui_h.at[pl.ds(base, 128)], idxv)
        g0 = pltpu.async_copy(ue_h.at[idxv], r0, sem)
        g1 = pltpu.async_copy(e1_h.at[idxv], r1, sem)
        g2 = pltpu.async_copy(e2_h.at[idxv], r2, sem)
        g0.wait(); g1.wait(); g2.wait()
        pltpu.sync_copy(r0, o_uego.at[pl.ds(base, 128)])

        def mean_body(i, carry):
            for q in range(4):
                a = r0[i, pl.ds(q * 16, 16)]
                bq = r1[i, pl.ds(q * 16, 16)]
                cq = r2[i, pl.ds(q * 16, 16)]
                r1[i, pl.ds(q * 16, 16)] = (a + bq + cq) * (1.0 / 3.0)
            return carry

        lax.fori_loop(0, 128, mean_body, 0)
        pltpu.sync_copy(r1, o_uemb.at[pl.ds(base, 128)])

        pltpu.sync_copy(pg_h.at[pl.ds(base, 128)], idxv)
        g0 = pltpu.async_copy(gf_h.at[idxv], r0, sem)
        g1 = pltpu.async_copy(ge_h.at[idxv], r1, sem)
        g0.wait(); g1.wait()
        pltpu.sync_copy(r0, o_pos.at[pl.ds(base, 128)])
        pltpu.sync_copy(r1, o_pego.at[pl.ds(base, 128)])

        pltpu.sync_copy(ng_h.at[pl.ds(base, 128)], idxv)
        g0 = pltpu.async_copy(gf_h.at[idxv], r0, sem)
        g1 = pltpu.async_copy(ge_h.at[idxv], r1, sem)
        g0.wait(); g1.wait()
        pltpu.sync_copy(r0, o_neg.at[pl.ds(base, 128)])
        pltpu.sync_copy(r1, o_nego.at[pl.ds(base, 128)])

    return k(user_emb, e1, e2, group_emb, gfin,
             user_inputs, pos_groups, neg_groups)


def _pack_chunks(arrs, n_tiles, loop_chunks, tch, pads):
    out = []
    for a, padv in zip(arrs, pads):
        n = n_tiles * loop_chunks * 128 - a.shape[0]
        ap = jnp.concatenate([a, jnp.full((n,), padv, a.dtype)])
        ap = ap.reshape(n_tiles, loop_chunks, 128)
        tail = jnp.full((n_tiles, tch - loop_chunks, 128), padv, a.dtype)
        out.append(jnp.concatenate([ap, tail], axis=1))
    return out


def _prep_hg_edges(row, col, val):
    c1 = col + NHG_PAD
    cp, c1p, rp, vp = _pack_chunks(
        [col, c1, row, val], NS, HG_LOOP, HG_TCH, [0, NHG_PAD, NHG_PAD, 0.0])
    meta = jnp.stack([cp, c1p, rp], axis=2)
    return meta.reshape(NS * HG_TCH * 3, 128), vp.reshape(NS * HG_TCH, 128)


def _prep_gi_edges(row, col, val):
    li = jnp.where(row < NG, row, NG)
    cp, lp, vp = _pack_chunks(
        [col, li, val], NC * NS, GI_LOOP, GI_TCH, [0, NG, 0.0])
    meta = jnp.stack([cp, lp], axis=2)
    return (meta.reshape(NC * NS * GI_TCH * 2, 128),
            vp.reshape(NC * NS * GI_TCH, 128))


def kernel(user_emb, item_emb, group_emb, hg_vals, gi_vals, gg_dense,
           hyper_W, hyper_b, light_W, light_b, over_W, over_b,
           hg_row, hg_col, gi_row, gi_col,
           user_inputs, pos_groups, neg_groups):
    x0 = jnp.concatenate([
        user_emb, group_emb,
        jnp.zeros((NHG_PAD - NHG, EMB), jnp.float32)], axis=0)
    x0cs = jnp.concatenate([x0[:, :32], x0[:, 32:]], axis=0)
    hmeta, hval = _prep_hg_edges(hg_row, hg_col, hg_vals)
    e1cs = _spmm_colsplit(x0cs, hmeta, hval, NHG_PAD, ACC_CS,
                          HG_LOOP, HG_TCH)
    e2cs = _spmm_colsplit(e1cs, hmeta, hval, NHG_PAD, ACC_CS,
                          HG_LOOP, HG_TCH)
    e1 = jnp.concatenate([e1cs[:NHG_PAD], e1cs[NHG_PAD:]], axis=1)
    e2 = jnp.concatenate([e2cs[:NHG_PAD], e2cs[NHG_PAD:]], axis=1)

    xg = jnp.concatenate([group_emb, item_emb], axis=0)
    gmeta, gval = _prep_gi_edges(gi_row, gi_col, gi_vals)
    gip = _spmm_gi(xg, gmeta, gval)

    wb = jnp.concatenate([
        jnp.concatenate([hyper_W, light_W, over_W], axis=1),
        jnp.stack([hyper_b[0], light_b[0], over_b[0]])[None, :],
    ], axis=0)
    gge = _ggmm_tc(gg_dense, group_emb)
    gfin = _combine_tc(gge, group_emb, e1, e2,
                       gip[:ACC_GI], gip[ACC_GI:], wb)

    return _batch_gather(user_emb, e1, e2, group_emb, gfin,
                         user_inputs, pos_groups, neg_groups)
